# baseline (speedup 1.0000x reference)
import functools

import jax
import jax.numpy as jnp
from jax import lax
from jax.experimental import pallas as pl
from jax.experimental.pallas import tpu as pltpu
from jax.experimental.pallas import tpu_sc as plsc

_NUM_ROWS = 10000
_H = 128
_B = 320000

_NC = 2
_NS = 16
_NW = _NC * _NS
_BPW = _B // _NW
_CHUNK = 80
_NFULL = _BPW // _CHUNK
_TAIL = _BPW - _NFULL * _CHUNK
_NBUF = 4
_NGROUPS = _NFULL // _NBUF
_NLEFT = _NFULL - _NGROUPS * _NBUF
_LANES = 16
_CGRP = _H // _LANES

_SCHUNKS = _NUM_ROWS // _CHUNK


def _gather(gate_theta, idx):
    mesh = plsc.VectorSubcoreMesh(core_axis_name="c", subcore_axis_name="s")

    @functools.partial(
        pl.kernel,
        mesh=mesh,
        out_type=jax.ShapeDtypeStruct((_B, _H), jnp.float32),
        scratch_types=(
            [pltpu.VMEM((_BPW,), jnp.int32)]
            + [pltpu.VMEM((_CHUNK, _H), jnp.float32)] * _NBUF
            + [pltpu.VMEM_SHARED((_NUM_ROWS, _H), jnp.float32)]
            + [pltpu.SemaphoreType.DMA] * (2 * _NBUF)
        ),
    )
    def k(table_hbm, idx_hbm, out_hbm, *scratch):
        idx_v = scratch[0]
        bufs = list(scratch[1:1 + _NBUF])
        tbl_sh = scratch[1 + _NBUF]
        gsems = list(scratch[2 + _NBUF:2 + 2 * _NBUF])
        ssems = list(scratch[2 + 2 * _NBUF:2 + 3 * _NBUF])
        c = lax.axis_index("c")
        s = lax.axis_index("s")
        wid = s * _NC + c
        base = wid * _BPW

        n_sig = jnp.where(s < _SCHUNKS - _NS * (_SCHUNKS // _NS),
                          _SCHUNKS // _NS + 1, _SCHUNKS // _NS)

        def sig_body(kk, carry):
            off = (kk * _NS + s) * _CHUNK
            pltpu.sync_copy(table_hbm.at[pl.ds(off, _CHUNK)], bufs[0])

            def row_body(r, carry2):
                for cg in range(_CGRP):
                    x = bufs[0][r, pl.ds(cg * _LANES, _LANES)]
                    bufs[0][r, pl.ds(cg * _LANES, _LANES)] = (
                        1.0 / (1.0 + jnp.exp(-x)))
                return carry2

            lax.fori_loop(0, _CHUNK, row_body, 0)
            pltpu.sync_copy(bufs[0], tbl_sh.at[pl.ds(off, _CHUNK)])
            return carry

        lax.fori_loop(0, n_sig, sig_body, 0)

        pltpu.sync_copy(idx_hbm.at[pl.ds(base, _BPW)], idx_v)
        plsc.subcore_barrier()

        def gather(off, b):
            return pltpu.async_copy(
                tbl_sh.at[idx_v.at[pl.ds(off, _CHUNK)]], bufs[b], gsems[b])

        def store(off, b):
            return pltpu.async_copy(
                bufs[b], out_hbm.at[pl.ds(base + off, _CHUNK)], ssems[b])

        def store_wait(b):
            pltpu.make_async_copy(
                bufs[b], out_hbm.at[pl.ds(base, _CHUNK)], ssems[b]).wait()

        prime = [gather(b * _CHUNK, b) for b in range(_NBUF)]
        for b in range(_NBUF):
            prime[b].wait()
            store(b * _CHUNK, b)

        def body(j, carry):
            handles = []
            for b in range(_NBUF):
                off = (_NBUF * j + b) * _CHUNK
                store_wait(b)
                handles.append(gather(off, b))
            for b in range(_NBUF):
                off = (_NBUF * j + b) * _CHUNK
                handles[b].wait()
                store(off, b)
            return carry

        lax.fori_loop(1, _NGROUPS, body, 0)

        left = []
        for b in range(_NLEFT):
            off = (_NGROUPS * _NBUF + b) * _CHUNK
            store_wait(b)
            left.append(gather(off, b))
        for b in range(_NLEFT):
            off = (_NGROUPS * _NBUF + b) * _CHUNK
            left[b].wait()
            store(off, b)

        if _TAIL:
            tb = _NLEFT
            ot = _NFULL * _CHUNK
            store_wait(tb)
            pltpu.async_copy(
                tbl_sh.at[idx_v.at[pl.ds(ot, _TAIL)]],
                bufs[tb].at[pl.ds(0, _TAIL)], gsems[tb]).wait()
            pltpu.sync_copy(bufs[tb].at[pl.ds(0, _TAIL)],
                            out_hbm.at[pl.ds(base + ot, _TAIL)])

        for b in range(_NBUF):
            if _TAIL and b == _NLEFT:
                continue
            store_wait(b)

    return k(gate_theta, idx)


def kernel(X, Y, gate_theta):
    idx = Y.astype(jnp.int32)
    return _gather(gate_theta, idx)


# speedup vs baseline: 1.1444x; 1.1444x over previous
import functools

import jax
import jax.numpy as jnp
from jax import lax
from jax.experimental import pallas as pl
from jax.experimental.pallas import tpu as pltpu
from jax.experimental.pallas import tpu_sc as plsc

_NUM_ROWS = 10000
_H = 128
_B = 320000

_NC = 2
_NS = 16
_NW = _NC * _NS
_BPW = _B // _NW
_CHUNK = 40
_NFULL = _BPW // _CHUNK
_TAIL = _BPW - _NFULL * _CHUNK
_NBUF = 8
_NGROUPS = _NFULL // _NBUF
_NLEFT = _NFULL - _NGROUPS * _NBUF
_RPT = 624
_RPT_REM = _NUM_ROWS - _NS * _RPT


def _sigmoid_body(t_ref, o_ref):
    o_ref[...] = jax.nn.sigmoid(t_ref[...])


def _sigmoid_table(gate_theta):
    return pl.pallas_call(
        _sigmoid_body,
        out_shape=jax.ShapeDtypeStruct((_NUM_ROWS, _H), jnp.float32),
    )(gate_theta)


def _gather(sig_table, idx):
    mesh = plsc.VectorSubcoreMesh(core_axis_name="c", subcore_axis_name="s")

    @functools.partial(
        pl.kernel,
        mesh=mesh,
        out_type=jax.ShapeDtypeStruct((_B, _H), jnp.float32),
        scratch_types=(
            [pltpu.VMEM((_BPW,), jnp.int32)]
            + [pltpu.VMEM((_CHUNK, _H), jnp.float32)] * _NBUF
            + [pltpu.VMEM_SHARED((_NUM_ROWS, _H), jnp.float32)]
            + [pltpu.SemaphoreType.DMA] * (2 * _NBUF)
        ),
    )
    def k(table_hbm, idx_hbm, out_hbm, *scratch):
        idx_v = scratch[0]
        bufs = list(scratch[1:1 + _NBUF])
        tbl_sh = scratch[1 + _NBUF]
        gsems = list(scratch[2 + _NBUF:2 + 2 * _NBUF])
        ssems = list(scratch[2 + 2 * _NBUF:2 + 3 * _NBUF])
        c = lax.axis_index("c")
        s = lax.axis_index("s")
        wid = s * _NC + c
        base = wid * _BPW

        pltpu.sync_copy(table_hbm.at[pl.ds(s * _RPT, _RPT)],
                        tbl_sh.at[pl.ds(s * _RPT, _RPT)])

        @pl.when(s == 0)
        def _stage_tail():
            pltpu.sync_copy(table_hbm.at[pl.ds(_NS * _RPT, _RPT_REM)],
                            tbl_sh.at[pl.ds(_NS * _RPT, _RPT_REM)])
        pltpu.sync_copy(idx_hbm.at[pl.ds(base, _BPW)], idx_v)
        plsc.subcore_barrier()

        def gather(off, b):
            return pltpu.async_copy(
                tbl_sh.at[idx_v.at[pl.ds(off, _CHUNK)]], bufs[b], gsems[b])

        def store(off, b):
            return pltpu.async_copy(
                bufs[b], out_hbm.at[pl.ds(base + off, _CHUNK)], ssems[b])

        def store_wait(b):
            pltpu.make_async_copy(
                bufs[b], out_hbm.at[pl.ds(base, _CHUNK)], ssems[b]).wait()

        prime = [gather(b * _CHUNK, b) for b in range(_NBUF)]
        for b in range(_NBUF):
            prime[b].wait()
            store(b * _CHUNK, b)

        def body(j, carry):
            handles = []
            for b in range(_NBUF):
                off = (_NBUF * j + b) * _CHUNK
                store_wait(b)
                handles.append(gather(off, b))
            for b in range(_NBUF):
                off = (_NBUF * j + b) * _CHUNK
                handles[b].wait()
                store(off, b)
            return carry

        lax.fori_loop(1, _NGROUPS, body, 0)

        left = []
        for b in range(_NLEFT):
            off = (_NGROUPS * _NBUF + b) * _CHUNK
            store_wait(b)
            left.append(gather(off, b))
        for b in range(_NLEFT):
            off = (_NGROUPS * _NBUF + b) * _CHUNK
            left[b].wait()
            store(off, b)

        if _TAIL:
            tb = _NLEFT
            ot = _NFULL * _CHUNK
            store_wait(tb)
            pltpu.async_copy(
                tbl_sh.at[idx_v.at[pl.ds(ot, _TAIL)]],
                bufs[tb].at[pl.ds(0, _TAIL)], gsems[tb]).wait()
            pltpu.sync_copy(bufs[tb].at[pl.ds(0, _TAIL)],
                            out_hbm.at[pl.ds(base + ot, _TAIL)])

        for b in range(_NBUF):
            if _TAIL and b == _NLEFT:
                continue
            store_wait(b)

    return k(sig_table, idx)


def kernel(X, Y, gate_theta):
    sig_table = _sigmoid_table(gate_theta)
    idx = Y.astype(jnp.int32)
    return _gather(sig_table, idx)
